# baseline (speedup 1.0000x reference)
import functools

import jax
import jax.numpy as jnp
from jax import lax
from jax.experimental import pallas as pl
from jax.experimental.pallas import tpu as pltpu
from jax.experimental.pallas import tpu_sc as plsc

_NC = 2
_NS = 16
_NW = _NC * _NS
_MESH = plsc.VectorSubcoreMesh(core_axis_name="c", subcore_axis_name="s")


@functools.lru_cache(maxsize=None)
def _stage_a(V, D, B0, S):
    n_tc = V // 128
    tail = V - n_tc * 128
    per_w = n_tc // _NW
    extra = n_tc - per_w * _NW
    bw = B0 // _NW

    @functools.partial(
        pl.kernel,
        mesh=_MESH,
        out_type=(
            jax.ShapeDtypeStruct((V * D,), jnp.float32),
            jax.ShapeDtypeStruct((B0 * S,), jnp.int32),
        ),
        compiler_params=pltpu.CompilerParams(needs_layout_passes=False),
        scratch_types=[
            pltpu.VMEM((S, bw), jnp.int32),
            pltpu.VMEM((D, 128), jnp.float32),
            pltpu.VMEM((D, 128), jnp.float32),
            pltpu.VMEM((4096,), jnp.float32),
            pltpu.VMEM((4096,), jnp.float32),
            pltpu.VMEM((4352,), jnp.float32),
            pltpu.VMEM((64, D), jnp.float32),
            pltpu.SemaphoreType.DMA,
            pltpu.SemaphoreType.DMA,
            pltpu.SemaphoreType.DMA,
            pltpu.SemaphoreType.DMA,
            pltpu.SemaphoreType.DMA,
        ],
    )
    def a(tt_hbm, xt_hbm, ttail_hbm, tlin_hbm, idx_hbm, xbuf, sbuf0, sbuf1,
          dbuf0, dbuf1, skew, tbuf, gsem0, gsem1, wsem0, wsem1, xsem):
        wid = lax.axis_index("s") * _NC + lax.axis_index("c")

        pltpu.sync_copy(xt_hbm.at[:, pl.ds(bw * wid, bw)], xbuf)
        for s in range(S):
            pltpu.make_async_copy(
                xbuf.at[s], idx_hbm.at[pl.ds(s * B0 + bw * wid, bw)], xsem
            ).start()

        iota = lax.iota(jnp.int32, 16)
        iota133 = iota * 133

        def tcol(t):
            return wid + t * _NW

        def fetch(t, buf, sem):
            return pltpu.make_async_copy(
                tt_hbm.at[:, pl.ds(tcol(t) * 128, 128)], buf, sem)

        def wout(t, buf, sem):
            return pltpu.make_async_copy(
                buf, tlin_hbm.at[pl.ds(tcol(t) * 4096, 4096)], sem)

        def transpose_block(src, dst):
            for d in range(D):
                for h in range(8):
                    vec = src[d, pl.ds(16 * h, 16)]
                    plsc.store_scatter(skew, [iota + (133 * d + 16 * h)], vec)
            for jp in range(32):
                for m in range(8):
                    vec = plsc.load_gather(
                        skew, [iota133 + (2128 * (m % 2) + 4 * jp + m // 2)])
                    dst[pl.ds(128 * jp + 16 * m, 16)] = vec

        n_t = per_w + 1
        n_mine = per_w + jnp.where(wid < extra, 1, 0)
        fetch(0, sbuf0, gsem0).start()

        def step(t, carry):
            for p in range(2):
                tt_i = 2 * t + p
                sem = gsem1 if p else gsem0
                wsem = wsem1 if p else wsem0
                nsem = gsem0 if p else gsem1
                sb = sbuf1 if p else sbuf0
                nsb = sbuf0 if p else sbuf1
                db = dbuf1 if p else dbuf0

                @pl.when(tt_i < n_mine)
                def _(tt_i=tt_i, sem=sem, wsem=wsem, nsem=nsem,
                      sb=sb, nsb=nsb, db=db):
                    @pl.when(tt_i + 1 < n_mine)
                    def _():
                        fetch(tt_i + 1, nsb, nsem).start()
                    fetch(tt_i, sb, sem).wait()
                    @pl.when(tt_i >= 2)
                    def _():
                        wout(tt_i - 2, db, wsem).wait()
                    transpose_block(sb, db)
                    wout(tt_i, db, wsem).start()
            return carry

        lax.fori_loop(0, (n_t + 1) // 2, step, 0)

        for p in range(2):
            wsem = wsem1 if p else wsem0
            db = dbuf1 if p else dbuf0
            pltpu.make_async_copy(
                db, tlin_hbm.at[pl.ds(0, 4096)], wsem).wait()

        @pl.when(wid == _NW - 1)
        def _():
            pltpu.sync_copy(ttail_hbm, tbuf)
            for vp in range(tail):
                for h2 in range(D // 16):
                    vec = tbuf[vp, pl.ds(16 * h2, 16)]
                    off = (vp // 4) * 128 + 32 * (vp % 4) + 16 * h2
                    plsc.store_scatter(dbuf0, [iota + off], vec)
            pltpu.sync_copy(dbuf0.at[pl.ds(0, tail * D)],
                            tlin_hbm.at[pl.ds(n_tc * 4096, tail * D)])

        for s in range(S):
            pltpu.make_async_copy(
                xbuf.at[0], idx_hbm.at[pl.ds(0, bw)], xsem).wait()

    return a


@functools.lru_cache(maxsize=None)
def _stage_b(B, V, D, chunk, K):
    b_per_w = B // _NW
    n_chunks = b_per_w // chunk
    R = n_chunks // K
    assert n_chunks % K == 0 and R % 2 == 0

    @functools.partial(
        pl.kernel,
        mesh=_MESH,
        out_type=jax.ShapeDtypeStruct((B, D), jnp.float32),
        compiler_params=pltpu.CompilerParams(use_tc_tiling_on_sc=False),
        scratch_types=[
            pltpu.VMEM((b_per_w,), jnp.int32),
            pltpu.VMEM((K, chunk, D), jnp.float32),
            pltpu.VMEM((K, chunk, D), jnp.float32),
            pltpu.SemaphoreType.DMA,
            pltpu.SemaphoreType.DMA,
            pltpu.SemaphoreType.DMA,
            pltpu.SemaphoreType.DMA,
        ],
    )
    def b(idx_hbm, table_hbm, out_hbm, idx_v, rows_a, rows_b,
          gsem_a, gsem_b, osem_a, osem_b):
        wid = lax.axis_index("s") * _NC + lax.axis_index("c")
        base = wid * b_per_w
        pltpu.sync_copy(idx_hbm.at[pl.ds(base, b_per_w)], idx_v)

        def gather(r, bank, bb, sem):
            off = pl.multiple_of((r * K + bb) * chunk, chunk)
            return pltpu.make_async_copy(
                table_hbm.at[idx_v.at[pl.ds(off, chunk)]], bank.at[bb], sem)

        def out_copy(r, bank, bb, sem):
            off = pl.multiple_of((r * K + bb) * chunk, chunk)
            return pltpu.make_async_copy(
                bank.at[bb], out_hbm.at[pl.ds(base + off, chunk)], sem)

        for bb in range(K):
            gather(0, rows_a, bb, gsem_a).start()

        def body(rp, carry):
            r0 = 2 * rp
            r1 = r0 + 1
            for bb in range(K):
                @pl.when(r0 > 0)
                def _(bb=bb):
                    out_copy(r1 - 2, rows_b, bb, osem_b).wait()
                gather(r1, rows_b, bb, gsem_b).start()
            for bb in range(K):
                gather(r0, rows_a, bb, gsem_a).wait()
                out_copy(r0, rows_a, bb, osem_a).start()
            for bb in range(K):
                @pl.when(r1 + 1 < R)
                def _(bb=bb):
                    out_copy(r0, rows_a, bb, osem_a).wait()
                    gather(r1 + 1, rows_a, bb, gsem_a).start()
            for bb in range(K):
                gather(r1, rows_b, bb, gsem_b).wait()
                out_copy(r1, rows_b, bb, osem_b).start()
            return carry

        lax.fori_loop(0, R // 2, body, 0)
        for bb in range(K):
            out_copy(R - 2, rows_a, bb, osem_a).wait()
            out_copy(R - 1, rows_b, bb, osem_b).wait()

    return b


@functools.lru_cache(maxsize=None)
def _stage_c(B0, S, D):
    n_blocks = S * (B0 // 128)
    per_w = n_blocks // _NW

    @functools.partial(
        pl.kernel,
        mesh=_MESH,
        out_type=jax.ShapeDtypeStruct((S, D, B0), jnp.float32),
        compiler_params=pltpu.CompilerParams(needs_layout_passes=False),
        scratch_types=[
            pltpu.VMEM((32, 128), jnp.float32),
            pltpu.VMEM((32, 128), jnp.float32),
            pltpu.VMEM((D, 128), jnp.float32),
            pltpu.VMEM((D, 128), jnp.float32),
            pltpu.VMEM((4352,), jnp.float32),
            pltpu.SemaphoreType.DMA,
            pltpu.SemaphoreType.DMA,
            pltpu.SemaphoreType.DMA,
            pltpu.SemaphoreType.DMA,
        ],
    )
    def c(gv_hbm, out_hbm, sbuf0, sbuf1, dbuf0, dbuf1, skew,
          gsem0, gsem1, wsem0, wsem1):
        wid = lax.axis_index("s") * _NC + lax.axis_index("c")
        beta0 = wid * per_w

        iota = lax.iota(jnp.int32, 16)
        kskew = (iota // 4) * 133 + (iota % 4) * 33

        def fetch(t, buf, sem):
            beta = beta0 + t
            s = beta // 128
            tc = beta % 128
            return pltpu.make_async_copy(
                gv_hbm.at[pl.ds(4096 * s + 32 * tc, 32), :], buf, sem)

        def wout(t, buf, sem):
            beta = beta0 + t
            s = beta // 128
            tc = beta % 128
            return pltpu.make_async_copy(
                buf, out_hbm.at[s, :, pl.ds(128 * tc, 128)], sem)

        def transpose_block(src, dst):
            for jp in range(32):
                for h in range(8):
                    vec = src[jp, pl.ds(16 * h, 16)]
                    off = 133 * jp + 33 * (h // 2) + 16 * (h % 2)
                    plsc.store_scatter(skew, [iota + off], vec)
            for d in range(D):
                for m in range(8):
                    vec = plsc.load_gather(skew, [kskew + (532 * m + d)])
                    dst[d, pl.ds(16 * m, 16)] = vec

        fetch(0, sbuf0, gsem0).start()

        def step(t, carry):
            for p in range(2):
                tt_i = 2 * t + p
                sem = gsem1 if p else gsem0
                wsem = wsem1 if p else wsem0
                nsem = gsem0 if p else gsem1
                sb = sbuf1 if p else sbuf0
                nsb = sbuf0 if p else sbuf1
                db = dbuf1 if p else dbuf0
                @pl.when(tt_i + 1 < per_w)
                def _(tt_i=tt_i, nsb=nsb, nsem=nsem):
                    fetch(tt_i + 1, nsb, nsem).start()
                fetch(tt_i, sb, sem).wait()
                @pl.when(tt_i >= 2)
                def _(tt_i=tt_i, db=db, wsem=wsem):
                    wout(tt_i - 2, db, wsem).wait()
                transpose_block(sb, db)
                wout(tt_i, db, wsem).start()
            return carry

        lax.fori_loop(0, per_w // 2, step, 0)
        for p in range(2):
            wsem = wsem1 if p else wsem0
            db = dbuf1 if p else dbuf0
            pltpu.make_async_copy(
                db, out_hbm.at[0, :, pl.ds(0, 128)], wsem).wait()

    return c


def kernel(x, table):
    B0, S = x.shape
    V, D = table.shape
    B = B0 * S
    tlin, idxr = _stage_a(V, D, B0, S)(table.T, x.T, table[V - (V % 128):])
    tfl2 = tlin.reshape(V, D)
    g = _stage_b(B, V, D, 256, 5)(idxr, tfl2)
    gv = g.reshape(B * D // 128, 128)
    out_t = _stage_c(B0, S, D)(gv)
    return out_t.transpose(2, 0, 1)


# speedup vs baseline: 1.0793x; 1.0793x over previous
import functools

import jax
import jax.numpy as jnp
from jax import lax
from jax.experimental import pallas as pl
from jax.experimental.pallas import tpu as pltpu
from jax.experimental.pallas import tpu_sc as plsc

_NC = 2
_NS = 16
_NW = _NC * _NS
_MESH = plsc.VectorSubcoreMesh(core_axis_name="c", subcore_axis_name="s")


@functools.lru_cache(maxsize=None)
def _stage_a(V, D, B0, S):
    FB = 1
    n_tc = V // 128
    tail = V - n_tc * 128
    n_sc = n_tc // FB
    per_w = n_sc // _NW
    extra = n_sc - per_w * _NW
    bw = B0 // _NW
    FW = FB * 128
    FD = FB * 4096

    @functools.partial(
        pl.kernel,
        mesh=_MESH,
        out_type=(
            jax.ShapeDtypeStruct((V * D,), jnp.float32),
            jax.ShapeDtypeStruct((B0 * S,), jnp.int32),
        ),
        compiler_params=pltpu.CompilerParams(needs_layout_passes=False),
        scratch_types=[
            pltpu.VMEM((S, bw), jnp.int32),
            pltpu.VMEM((D, FW), jnp.float32),
            pltpu.VMEM((D, FW), jnp.float32),
            pltpu.VMEM((FD,), jnp.float32),
            pltpu.VMEM((FD,), jnp.float32),
            pltpu.VMEM((64, D), jnp.float32),
            pltpu.SemaphoreType.DMA,
            pltpu.SemaphoreType.DMA,
            pltpu.SemaphoreType.DMA,
            pltpu.SemaphoreType.DMA,
            pltpu.SemaphoreType.DMA,
        ],
    )
    def a(tt_hbm, xt_hbm, ttail_hbm, tlin_hbm, idx_hbm, xbuf, sbuf0, sbuf1,
          dbuf0, dbuf1, tbuf, gsem0, gsem1, wsem0, wsem1, xsem):
        wid = lax.axis_index("s") * _NC + lax.axis_index("c")

        pltpu.sync_copy(xt_hbm.at[:, pl.ds(bw * wid, bw)], xbuf)
        for s in range(S):
            pltpu.make_async_copy(
                xbuf.at[s], idx_hbm.at[pl.ds(s * B0 + bw * wid, bw)], xsem
            ).start()

        iota = lax.iota(jnp.int32, 16)
        pbase = (iota // 4) * 128 + (iota % 4) * 32

        def scol(t):
            return wid + t * _NW

        def fetch(t, buf, sem):
            return pltpu.make_async_copy(
                tt_hbm.at[:, pl.ds(scol(t) * FW, FW)], buf, sem)

        def wout(t, buf, sem):
            return pltpu.make_async_copy(
                buf, tlin_hbm.at[pl.ds(scol(t) * FD, FD)], sem)

        def transpose_block(src, dst):
            for sub in range(FB):
                for d in range(D):
                    for h in range(8):
                        vec = src[d, pl.ds(128 * sub + 16 * h, 16)]
                        plsc.store_scatter(
                            dst, [pbase + (4096 * sub + 512 * h + d)], vec)

        n_t = per_w + 1
        n_mine = per_w + jnp.where(wid < extra, 1, 0)
        fetch(0, sbuf0, gsem0).start()

        def step(t, carry):
            for p in range(2):
                tt_i = 2 * t + p
                sem = gsem1 if p else gsem0
                wsem = wsem1 if p else wsem0
                nsem = gsem0 if p else gsem1
                sb = sbuf1 if p else sbuf0
                nsb = sbuf0 if p else sbuf1
                db = dbuf1 if p else dbuf0

                @pl.when(tt_i < n_mine)
                def _(tt_i=tt_i, sem=sem, wsem=wsem, nsem=nsem,
                      sb=sb, nsb=nsb, db=db):
                    @pl.when(tt_i + 1 < n_mine)
                    def _():
                        fetch(tt_i + 1, nsb, nsem).start()
                    fetch(tt_i, sb, sem).wait()
                    @pl.when(tt_i >= 2)
                    def _():
                        wout(tt_i - 2, db, wsem).wait()
                    transpose_block(sb, db)
                    wout(tt_i, db, wsem).start()
            return carry

        lax.fori_loop(0, (n_t + 1) // 2, step, 0)

        for p in range(2):
            wsem = wsem1 if p else wsem0
            db = dbuf1 if p else dbuf0
            pltpu.make_async_copy(
                db, tlin_hbm.at[pl.ds(0, FD)], wsem).wait()

        @pl.when(wid == _NW - 1)
        def _():
            pltpu.sync_copy(ttail_hbm, tbuf)
            for vp in range(tail):
                for h2 in range(D // 16):
                    vec = tbuf[vp, pl.ds(16 * h2, 16)]
                    off = (vp // 4) * 128 + 32 * (vp % 4) + 16 * h2
                    plsc.store_scatter(dbuf0, [iota + off], vec)
            pltpu.sync_copy(dbuf0.at[pl.ds(0, tail * D)],
                            tlin_hbm.at[pl.ds(n_tc * 4096, tail * D)])

        for s in range(S):
            pltpu.make_async_copy(
                xbuf.at[0], idx_hbm.at[pl.ds(0, bw)], xsem).wait()

    return a


@functools.lru_cache(maxsize=None)
def _stage_b(B, V, D, chunk, K):
    b_per_w = B // _NW
    n_chunks = b_per_w // chunk
    R = n_chunks // K
    assert n_chunks % K == 0 and R % 2 == 0

    @functools.partial(
        pl.kernel,
        mesh=_MESH,
        out_type=jax.ShapeDtypeStruct((B, D), jnp.float32),
        compiler_params=pltpu.CompilerParams(use_tc_tiling_on_sc=False),
        scratch_types=[
            pltpu.VMEM((b_per_w,), jnp.int32),
            pltpu.VMEM((K, chunk, D), jnp.float32),
            pltpu.VMEM((K, chunk, D), jnp.float32),
            pltpu.SemaphoreType.DMA,
            pltpu.SemaphoreType.DMA,
            pltpu.SemaphoreType.DMA,
            pltpu.SemaphoreType.DMA,
        ],
    )
    def b(idx_hbm, table_hbm, out_hbm, idx_v, rows_a, rows_b,
          gsem_a, gsem_b, osem_a, osem_b):
        wid = lax.axis_index("s") * _NC + lax.axis_index("c")
        base = wid * b_per_w
        pltpu.sync_copy(idx_hbm.at[pl.ds(base, b_per_w)], idx_v)

        def gather(r, bank, bb, sem):
            off = pl.multiple_of((r * K + bb) * chunk, chunk)
            return pltpu.make_async_copy(
                table_hbm.at[idx_v.at[pl.ds(off, chunk)]], bank.at[bb], sem)

        def out_copy(r, bank, bb, sem):
            off = pl.multiple_of((r * K + bb) * chunk, chunk)
            return pltpu.make_async_copy(
                bank.at[bb], out_hbm.at[pl.ds(base + off, chunk)], sem)

        for bb in range(K):
            gather(0, rows_a, bb, gsem_a).start()

        def body(rp, carry):
            r0 = 2 * rp
            r1 = r0 + 1
            for bb in range(K):
                @pl.when(r0 > 0)
                def _(bb=bb):
                    out_copy(r1 - 2, rows_b, bb, osem_b).wait()
                gather(r1, rows_b, bb, gsem_b).start()
            for bb in range(K):
                gather(r0, rows_a, bb, gsem_a).wait()
                out_copy(r0, rows_a, bb, osem_a).start()
            for bb in range(K):
                @pl.when(r1 + 1 < R)
                def _(bb=bb):
                    out_copy(r0, rows_a, bb, osem_a).wait()
                    gather(r1 + 1, rows_a, bb, gsem_a).start()
            for bb in range(K):
                gather(r1, rows_b, bb, gsem_b).wait()
                out_copy(r1, rows_b, bb, osem_b).start()
            return carry

        lax.fori_loop(0, R // 2, body, 0)
        for bb in range(K):
            out_copy(R - 2, rows_a, bb, osem_a).wait()
            out_copy(R - 1, rows_b, bb, osem_b).wait()

    return b


@functools.lru_cache(maxsize=None)
def _stage_c(B0, S, D):
    FB = 1
    n_sb = S * (B0 // (128 * FB))
    per_w = n_sb // _NW
    NTC = B0 // (128 * FB)

    @functools.partial(
        pl.kernel,
        mesh=_MESH,
        out_type=jax.ShapeDtypeStruct((S, D, B0), jnp.float32),
        compiler_params=pltpu.CompilerParams(needs_layout_passes=False),
        scratch_types=[
            pltpu.VMEM((32 * FB, 128), jnp.float32),
            pltpu.VMEM((32 * FB, 128), jnp.float32),
            pltpu.VMEM((D, 128 * FB), jnp.float32),
            pltpu.VMEM((D, 128 * FB), jnp.float32),
            pltpu.SemaphoreType.DMA,
            pltpu.SemaphoreType.DMA,
            pltpu.SemaphoreType.DMA,
            pltpu.SemaphoreType.DMA,
        ],
    )
    def c(gv_hbm, out_hbm, sbuf0, sbuf1, dbuf0, dbuf1,
          gsem0, gsem1, wsem0, wsem1):
        wid = lax.axis_index("s") * _NC + lax.axis_index("c")
        sb0 = wid * per_w

        iota = lax.iota(jnp.int32, 16)
        row0 = iota
        row1 = iota + 16

        def fetch(t, buf, sem):
            sig = sb0 + t
            s = sig // NTC
            tc4 = sig % NTC
            return pltpu.make_async_copy(
                gv_hbm.at[pl.ds(4096 * s + 32 * FB * tc4, 32 * FB), :],
                buf, sem)

        def wout(t, buf, sem):
            sig = sb0 + t
            s = sig // NTC
            tc4 = sig % NTC
            return pltpu.make_async_copy(
                buf, out_hbm.at[s, :, pl.ds(128 * FB * tc4, 128 * FB)], sem)

        def transpose_block(src, dst):
            for sub in range(FB):
                for jp in range(32):
                    for h in range(8):
                        vec = src[32 * sub + jp, pl.ds(16 * h, 16)]
                        idx_r = row1 if (h % 2) else row0
                        idx_l = jnp.full(
                            (16,), 128 * sub + 4 * jp + h // 2, jnp.int32)
                        plsc.store_scatter(dst, [idx_r, idx_l], vec)

        fetch(0, sbuf0, gsem0).start()

        def step(t, carry):
            for p in range(2):
                tt_i = 2 * t + p
                sem = gsem1 if p else gsem0
                wsem = wsem1 if p else wsem0
                nsem = gsem0 if p else gsem1
                sb = sbuf1 if p else sbuf0
                nsb = sbuf0 if p else sbuf1
                db = dbuf1 if p else dbuf0
                @pl.when(tt_i + 1 < per_w)
                def _(tt_i=tt_i, nsb=nsb, nsem=nsem):
                    fetch(tt_i + 1, nsb, nsem).start()
                fetch(tt_i, sb, sem).wait()
                @pl.when(tt_i >= 2)
                def _(tt_i=tt_i, db=db, wsem=wsem):
                    wout(tt_i - 2, db, wsem).wait()
                transpose_block(sb, db)
                wout(tt_i, db, wsem).start()
            return carry

        lax.fori_loop(0, per_w // 2, step, 0)
        for p in range(2):
            wsem = wsem1 if p else wsem0
            db = dbuf1 if p else dbuf0
            pltpu.make_async_copy(
                db, out_hbm.at[0, :, pl.ds(0, 128 * FB)], wsem).wait()

    return c


def kernel(x, table):
    B0, S = x.shape
    V, D = table.shape
    B = B0 * S
    tlin, idxr = _stage_a(V, D, B0, S)(table.T, x.T, table[V - (V % 128):])
    tfl2 = tlin.reshape(V, D)
    g = _stage_b(B, V, D, 256, 5)(idxr, tfl2)
    gv = g.reshape(B * D // 128, 128)
    out_t = _stage_c(B0, S, D)(gv)
    return out_t.transpose(2, 0, 1)
